# baseline (speedup 1.0000x reference)
import jax
import jax.numpy as jnp
from jax import lax
from jax.experimental import pallas as pl
from jax.experimental.pallas import tpu as pltpu
from jax.experimental.pallas import tpu_sc as plsc

N = 10000
D = 128
E = 320000
NC = 2
NS = 16
NW = NC * NS

CH = 128

AGG_ROWS = E // CH
AGG_BASE = AGG_ROWS // NW
AGG_EXTRA = AGG_ROWS - AGG_BASE * NW

ROWS_PER_TILE = N // NS
DEG_W = 16

RING = 8
LOOKAHEAD = 4
MAIN = (AGG_BASE // RING) * RING


def _mesh():
    return plsc.VectorSubcoreMesh(core_axis_name="c", subcore_axis_name="s")


_sc_params = pltpu.CompilerParams(use_tc_tiling_on_sc=False)


def _deg_body(er_hbm, upd_hbm, zero_hbm, out_hbm,
              sidx, didx, ubuf, acc_s, acc_d, sem):
    c = lax.axis_index("c")
    s = lax.axis_index("s")
    w = c * NS + s

    pltpu.sync_copy(er_hbm.at[0, pl.ds(w * AGG_BASE, AGG_BASE)],
                    sidx.at[pl.ds(0, AGG_BASE)])
    pltpu.sync_copy(er_hbm.at[1, pl.ds(w * AGG_BASE, AGG_BASE)],
                    didx.at[pl.ds(0, AGG_BASE)])

    @pl.when(w < AGG_EXTRA)
    def _():
        pltpu.sync_copy(er_hbm.at[0, pl.ds(NW * AGG_BASE + w, 1)],
                        sidx.at[pl.ds(AGG_BASE, 1)])
        pltpu.sync_copy(er_hbm.at[1, pl.ds(NW * AGG_BASE + w, 1)],
                        didx.at[pl.ds(AGG_BASE, 1)])

    pltpu.sync_copy(upd_hbm, ubuf)

    pltpu.sync_copy(zero_hbm, acc_s.at[pl.ds(s * ROWS_PER_TILE, ROWS_PER_TILE)])
    pltpu.sync_copy(zero_hbm, acc_d.at[pl.ds(s * ROWS_PER_TILE, ROWS_PER_TILE)])
    plsc.subcore_barrier()

    @pl.loop(0, AGG_BASE // 3)
    def _(jj):
        j = jj * 3
        pltpu.async_copy(ubuf, acc_s.at[sidx.at[j]], sem, add=True)
        pltpu.async_copy(ubuf, acc_d.at[didx.at[j]], sem, add=True)
        pltpu.async_copy(ubuf, acc_s.at[sidx.at[j + 1]], sem, add=True)
        pltpu.async_copy(ubuf, acc_d.at[didx.at[j + 1]], sem, add=True)
        pltpu.async_copy(ubuf, acc_s.at[sidx.at[j + 2]], sem, add=True)
        pltpu.async_copy(ubuf, acc_d.at[didx.at[j + 2]], sem, add=True)
        pltpu.make_async_copy(ubuf, acc_s.at[sidx.at[j]], sem).wait()
        pltpu.make_async_copy(ubuf, acc_d.at[didx.at[j]], sem).wait()
        pltpu.make_async_copy(ubuf, acc_s.at[sidx.at[j + 1]], sem).wait()
        pltpu.make_async_copy(ubuf, acc_d.at[didx.at[j + 1]], sem).wait()
        pltpu.make_async_copy(ubuf, acc_s.at[sidx.at[j + 2]], sem).wait()
        pltpu.make_async_copy(ubuf, acc_d.at[didx.at[j + 2]], sem).wait()

    @pl.when(w < AGG_EXTRA)
    def _():
        pltpu.sync_copy(ubuf, acc_s.at[sidx.at[AGG_BASE]], add=True)
        pltpu.sync_copy(ubuf, acc_d.at[didx.at[AGG_BASE]], add=True)

    plsc.subcore_barrier()

    base = s * ROWS_PER_TILE
    pltpu.sync_copy(acc_s.at[pl.ds(base, ROWS_PER_TILE)],
                    out_hbm.at[pl.ds(c * 2 * N + base, ROWS_PER_TILE)])
    pltpu.sync_copy(acc_d.at[pl.ds(base, ROWS_PER_TILE)],
                    out_hbm.at[pl.ds(c * 2 * N + N + base, ROWS_PER_TILE)])


_deg_call = pl.kernel(
    _deg_body,
    out_type=jax.ShapeDtypeStruct((2 * 2 * N, DEG_W), jnp.float32),
    mesh=_mesh(),
    compiler_params=_sc_params,
    scratch_types=[
        pltpu.VMEM((AGG_BASE + 1, CH), jnp.int32),
        pltpu.VMEM((AGG_BASE + 1, CH), jnp.int32),
        pltpu.VMEM((CH, DEG_W), jnp.float32),
        pltpu.VMEM_SHARED((N, DEG_W), jnp.float32),
        pltpu.VMEM_SHARED((N, DEG_W), jnp.float32),
        pltpu.SemaphoreType.DMA,
    ],
)


def _agg_body(g_hbm, er_hbm, zero_hbm, out_hbm,
              sidx, didx, r0, r1, r2, r3, r4, r5, r6, r7, acc_sh,
              sem_g, sem_s):
    c = lax.axis_index("c")
    s = lax.axis_index("s")
    w = c * NS + s
    rows = [r0, r1, r2, r3, r4, r5, r6, r7]

    pltpu.sync_copy(zero_hbm, acc_sh.at[pl.ds(s * ROWS_PER_TILE, ROWS_PER_TILE)])
    plsc.subcore_barrier()

    pltpu.sync_copy(er_hbm.at[0, pl.ds(w * AGG_BASE, AGG_BASE)],
                    sidx.at[pl.ds(0, AGG_BASE)])
    pltpu.sync_copy(er_hbm.at[1, pl.ds(w * AGG_BASE, AGG_BASE)],
                    didx.at[pl.ds(0, AGG_BASE)])

    @pl.when(w < AGG_EXTRA)
    def _():
        pltpu.sync_copy(er_hbm.at[0, pl.ds(NW * AGG_BASE + w, 1)],
                        sidx.at[pl.ds(AGG_BASE, 1)])
        pltpu.sync_copy(er_hbm.at[1, pl.ds(NW * AGG_BASE + w, 1)],
                        didx.at[pl.ds(AGG_BASE, 1)])

    for t in range(LOOKAHEAD):
        pltpu.async_copy(g_hbm.at[sidx.at[t]], rows[t], sem_g)

    def _wait_scatter(b):
        pltpu.make_async_copy(rows[b], acc_sh.at[didx.at[0]], sem_s).wait()

    def step(j, b, first_octet):
        nb = (b + LOOKAHEAD) % RING
        if first_octet:
            @pl.when(j >= LOOKAHEAD)
            def _():
                _wait_scatter(nb)
        else:
            _wait_scatter(nb)

        @pl.when(j + LOOKAHEAD < AGG_BASE)
        def _():
            pltpu.async_copy(g_hbm.at[sidx.at[j + LOOKAHEAD]], rows[nb], sem_g)

        pltpu.make_async_copy(g_hbm.at[sidx.at[j]], rows[b], sem_g).wait()
        pltpu.async_copy(rows[b], acc_sh.at[didx.at[j]], sem_s, add=True)

    @pl.loop(0, MAIN // RING)
    def _(o):
        for t in range(RING):
            step(o * RING + t, t, first_octet=True)

    for t in range(AGG_BASE - MAIN):
        step(MAIN + t, t, first_octet=False)

    for t in range(LOOKAHEAD):
        _wait_scatter((AGG_BASE - LOOKAHEAD + t) % RING)

    @pl.when(w < AGG_EXTRA)
    def _():
        pltpu.async_copy(g_hbm.at[sidx.at[AGG_BASE]], r0, sem_g).wait()
        pltpu.sync_copy(r0, acc_sh.at[didx.at[AGG_BASE]], add=True)

    plsc.subcore_barrier()

    base = s * ROWS_PER_TILE
    pltpu.sync_copy(acc_sh.at[pl.ds(base, ROWS_PER_TILE)],
                    out_hbm.at[pl.ds(c * N + base, ROWS_PER_TILE)])


_agg_call = pl.kernel(
    _agg_body,
    out_type=jax.ShapeDtypeStruct((2 * N, D), jnp.bfloat16),
    mesh=_mesh(),
    compiler_params=_sc_params,
    scratch_types=[
        pltpu.VMEM((AGG_BASE + 1, CH), jnp.int32),
        pltpu.VMEM((AGG_BASE + 1, CH), jnp.int32),
        pltpu.VMEM((CH, D), jnp.bfloat16),
        pltpu.VMEM((CH, D), jnp.bfloat16),
        pltpu.VMEM((CH, D), jnp.bfloat16),
        pltpu.VMEM((CH, D), jnp.bfloat16),
        pltpu.VMEM((CH, D), jnp.bfloat16),
        pltpu.VMEM((CH, D), jnp.bfloat16),
        pltpu.VMEM((CH, D), jnp.bfloat16),
        pltpu.VMEM((CH, D), jnp.bfloat16),
        pltpu.VMEM_SHARED((N, D), jnp.bfloat16),
        pltpu.SemaphoreType.DMA,
        pltpu.SemaphoreType.DMA,
    ],
)


RB = 1000


def _norm(a_ref, b_ref):
    return lax.rsqrt(a_ref[:, :1] + b_ref[:, :1] + 1.0)


def _tc1_body(x_ref, ns_ref, w_ref, o_ref):
    o_ref[...] = jnp.dot(x_ref[...] * ns_ref[...], w_ref[...],
                         preferred_element_type=jnp.float32).astype(jnp.bfloat16)


def _tc2_body(p0_ref, p1_ref, g_ref, nd_ref, ns_ref, b_ref, w_ref, o_ref):
    h = ((p0_ref[...] + p1_ref[...]).astype(jnp.float32)
         + g_ref[...].astype(jnp.float32)) * nd_ref[...] + b_ref[...]
    h = jnp.maximum(h, 0.0)
    o_ref[...] = jnp.dot(h * ns_ref[...], w_ref[...],
                         preferred_element_type=jnp.float32).astype(jnp.bfloat16)


def _tc3_body(p0_ref, p1_ref, g_ref, nd_ref, b_ref, o_ref):
    o_ref[...] = ((p0_ref[...] + p1_ref[...]).astype(jnp.float32)
                  + g_ref[...].astype(jnp.float32)) * nd_ref[...] + b_ref[...]


_NBLOCKS = N // RB

_mat_spec = pl.BlockSpec((RB, D), lambda i: (i, 0))
_vec_spec = pl.BlockSpec((RB, 1), lambda i: (i, 0))
_w_spec = pl.BlockSpec((D, D), lambda i: (0, 0))
_b_spec = pl.BlockSpec((1, D), lambda i: (0, 0))

_tc1 = pl.pallas_call(
    _tc1_body,
    out_shape=jax.ShapeDtypeStruct((N, D), jnp.bfloat16),
    grid=(_NBLOCKS,),
    in_specs=[_mat_spec, _vec_spec, _w_spec],
    out_specs=_mat_spec,
)

_tc2 = pl.pallas_call(
    _tc2_body,
    out_shape=jax.ShapeDtypeStruct((N, D), jnp.bfloat16),
    grid=(_NBLOCKS,),
    in_specs=[_mat_spec, _mat_spec, _mat_spec, _vec_spec, _vec_spec,
              _b_spec, _w_spec],
    out_specs=_mat_spec,
)

_tc3 = pl.pallas_call(
    _tc3_body,
    out_shape=jax.ShapeDtypeStruct((N, D), jnp.float32),
    grid=(_NBLOCKS,),
    in_specs=[_mat_spec, _mat_spec, _mat_spec, _vec_spec, _b_spec],
    out_specs=_mat_spec,
)


def kernel(x, edge_index, W1, b1, W2, b2):
    er = edge_index.reshape(2, AGG_ROWS, CH)
    upd = jnp.zeros((CH, DEG_W), jnp.float32).at[:, 0].set(1.0)
    zdeg = jnp.zeros((ROWS_PER_TILE, DEG_W), jnp.float32)
    zagg = jnp.zeros((ROWS_PER_TILE, D), jnp.bfloat16)

    degp = _deg_call(er, upd, zdeg)
    dcol = degp[:, 0]
    ns = lax.rsqrt(dcol[:N] + dcol[2 * N : 3 * N] + 1.0)[:, None]
    nd = lax.rsqrt(dcol[N : 2 * N] + dcol[3 * N :] + 1.0)[:, None]

    g1 = _tc1(x, ns, W1)
    p1 = _agg_call(g1, er, zagg)
    g2 = _tc2(p1[:N], p1[N:], g1, nd, ns, b1.reshape(1, D), W2)
    p2 = _agg_call(g2, er, zagg)
    out = _tc3(p2[:N], p2[N:], g2, nd, b2.reshape(1, D))
    return out


# speedup vs baseline: 1.1032x; 1.1032x over previous
import jax
import jax.numpy as jnp
from jax import lax
from jax.experimental import pallas as pl
from jax.experimental.pallas import tpu as pltpu
from jax.experimental.pallas import tpu_sc as plsc

N = 10000
D = 128
E = 320000
NC = 2
NS = 16
NW = NC * NS

CH = 128

AGG_ROWS = E // CH
AGG_BASE = AGG_ROWS // NW
AGG_EXTRA = AGG_ROWS - AGG_BASE * NW

ROWS_PER_TILE = N // NS
DEG_W = 16

RING = 8
LOOKAHEAD = 4
MAIN = (AGG_BASE // RING) * RING


def _mesh():
    return plsc.VectorSubcoreMesh(core_axis_name="c", subcore_axis_name="s")


_sc_params = pltpu.CompilerParams(use_tc_tiling_on_sc=False)


def _deg_body(er_hbm, upd_hbm, zero_hbm, out_hbm,
              sidx, didx, ubuf, acc_s, acc_d, sem):
    c = lax.axis_index("c")
    s = lax.axis_index("s")
    w = c * NS + s

    pltpu.sync_copy(er_hbm.at[0, pl.ds(w * AGG_BASE, AGG_BASE)],
                    sidx.at[pl.ds(0, AGG_BASE)])
    pltpu.sync_copy(er_hbm.at[1, pl.ds(w * AGG_BASE, AGG_BASE)],
                    didx.at[pl.ds(0, AGG_BASE)])

    @pl.when(w < AGG_EXTRA)
    def _():
        pltpu.sync_copy(er_hbm.at[0, pl.ds(NW * AGG_BASE + w, 1)],
                        sidx.at[pl.ds(AGG_BASE, 1)])
        pltpu.sync_copy(er_hbm.at[1, pl.ds(NW * AGG_BASE + w, 1)],
                        didx.at[pl.ds(AGG_BASE, 1)])

    pltpu.sync_copy(upd_hbm, ubuf)

    pltpu.sync_copy(zero_hbm, acc_s.at[pl.ds(s * ROWS_PER_TILE, ROWS_PER_TILE)])
    pltpu.sync_copy(zero_hbm, acc_d.at[pl.ds(s * ROWS_PER_TILE, ROWS_PER_TILE)])
    plsc.subcore_barrier()

    @pl.loop(0, AGG_BASE // 3)
    def _(jj):
        j = jj * 3
        pltpu.async_copy(ubuf, acc_s.at[sidx.at[j]], sem, add=True)
        pltpu.async_copy(ubuf, acc_d.at[didx.at[j]], sem, add=True)
        pltpu.async_copy(ubuf, acc_s.at[sidx.at[j + 1]], sem, add=True)
        pltpu.async_copy(ubuf, acc_d.at[didx.at[j + 1]], sem, add=True)
        pltpu.async_copy(ubuf, acc_s.at[sidx.at[j + 2]], sem, add=True)
        pltpu.async_copy(ubuf, acc_d.at[didx.at[j + 2]], sem, add=True)
        pltpu.make_async_copy(ubuf, acc_s.at[sidx.at[j]], sem).wait()
        pltpu.make_async_copy(ubuf, acc_d.at[didx.at[j]], sem).wait()
        pltpu.make_async_copy(ubuf, acc_s.at[sidx.at[j + 1]], sem).wait()
        pltpu.make_async_copy(ubuf, acc_d.at[didx.at[j + 1]], sem).wait()
        pltpu.make_async_copy(ubuf, acc_s.at[sidx.at[j + 2]], sem).wait()
        pltpu.make_async_copy(ubuf, acc_d.at[didx.at[j + 2]], sem).wait()

    @pl.when(w < AGG_EXTRA)
    def _():
        pltpu.sync_copy(ubuf, acc_s.at[sidx.at[AGG_BASE]], add=True)
        pltpu.sync_copy(ubuf, acc_d.at[didx.at[AGG_BASE]], add=True)

    plsc.subcore_barrier()

    base = s * ROWS_PER_TILE
    pltpu.sync_copy(acc_s.at[pl.ds(base, ROWS_PER_TILE)],
                    out_hbm.at[pl.ds(c * 2 * N + base, ROWS_PER_TILE)])
    pltpu.sync_copy(acc_d.at[pl.ds(base, ROWS_PER_TILE)],
                    out_hbm.at[pl.ds(c * 2 * N + N + base, ROWS_PER_TILE)])


_deg_call = pl.kernel(
    _deg_body,
    out_type=jax.ShapeDtypeStruct((2 * 2 * N, DEG_W), jnp.float32),
    mesh=_mesh(),
    compiler_params=_sc_params,
    scratch_types=[
        pltpu.VMEM((AGG_BASE + 1, CH), jnp.int32),
        pltpu.VMEM((AGG_BASE + 1, CH), jnp.int32),
        pltpu.VMEM((CH, DEG_W), jnp.float32),
        pltpu.VMEM_SHARED((N, DEG_W), jnp.float32),
        pltpu.VMEM_SHARED((N, DEG_W), jnp.float32),
        pltpu.SemaphoreType.DMA,
    ],
)


def _agg_body(g_hbm, er_hbm, zero_hbm, out_hbm,
              sidx, didx, r0, r1, r2, r3, r4, r5, r6, r7, acc_sh,
              sem_g, sem_s):
    c = lax.axis_index("c")
    s = lax.axis_index("s")
    w = c * NS + s
    rows = [r0, r1, r2, r3, r4, r5, r6, r7]

    pltpu.sync_copy(zero_hbm, acc_sh.at[pl.ds(s * ROWS_PER_TILE, ROWS_PER_TILE)])
    plsc.subcore_barrier()

    pltpu.sync_copy(er_hbm.at[0, pl.ds(w * AGG_BASE, AGG_BASE)],
                    sidx.at[pl.ds(0, AGG_BASE)])
    pltpu.sync_copy(er_hbm.at[1, pl.ds(w * AGG_BASE, AGG_BASE)],
                    didx.at[pl.ds(0, AGG_BASE)])

    @pl.when(w < AGG_EXTRA)
    def _():
        pltpu.sync_copy(er_hbm.at[0, pl.ds(NW * AGG_BASE + w, 1)],
                        sidx.at[pl.ds(AGG_BASE, 1)])
        pltpu.sync_copy(er_hbm.at[1, pl.ds(NW * AGG_BASE + w, 1)],
                        didx.at[pl.ds(AGG_BASE, 1)])

    for t in range(LOOKAHEAD):
        pltpu.async_copy(g_hbm.at[sidx.at[t]], rows[t], sem_g)

    def _wait_scatter(b):
        pltpu.make_async_copy(rows[b], acc_sh.at[didx.at[0]], sem_s).wait()

    def step(j, b, first_octet):
        nb = (b + LOOKAHEAD) % RING
        if first_octet:
            @pl.when(j >= LOOKAHEAD)
            def _():
                _wait_scatter(nb)
        else:
            _wait_scatter(nb)

        @pl.when(j + LOOKAHEAD < AGG_BASE)
        def _():
            pltpu.async_copy(g_hbm.at[sidx.at[j + LOOKAHEAD]], rows[nb], sem_g)

        pltpu.make_async_copy(g_hbm.at[sidx.at[j]], rows[b], sem_g).wait()
        pltpu.async_copy(rows[b], acc_sh.at[didx.at[j]], sem_s, add=True)

    @pl.loop(0, MAIN // RING)
    def _(o):
        for t in range(RING):
            step(o * RING + t, t, first_octet=True)

    for t in range(AGG_BASE - MAIN):
        step(MAIN + t, t, first_octet=False)

    for t in range(LOOKAHEAD):
        _wait_scatter((AGG_BASE - LOOKAHEAD + t) % RING)

    @pl.when(w < AGG_EXTRA)
    def _():
        pltpu.async_copy(g_hbm.at[sidx.at[AGG_BASE]], r0, sem_g).wait()
        pltpu.sync_copy(r0, acc_sh.at[didx.at[AGG_BASE]], add=True)

    plsc.subcore_barrier()

    base = s * ROWS_PER_TILE
    pltpu.sync_copy(acc_sh.at[pl.ds(base, ROWS_PER_TILE)],
                    out_hbm.at[pl.ds(c * N + base, ROWS_PER_TILE)])


_agg_call = pl.kernel(
    _agg_body,
    out_type=jax.ShapeDtypeStruct((2 * N, D), jnp.bfloat16),
    mesh=_mesh(),
    compiler_params=_sc_params,
    scratch_types=[
        pltpu.VMEM((AGG_BASE + 1, CH), jnp.int32),
        pltpu.VMEM((AGG_BASE + 1, CH), jnp.int32),
        pltpu.VMEM((CH, D), jnp.bfloat16),
        pltpu.VMEM((CH, D), jnp.bfloat16),
        pltpu.VMEM((CH, D), jnp.bfloat16),
        pltpu.VMEM((CH, D), jnp.bfloat16),
        pltpu.VMEM((CH, D), jnp.bfloat16),
        pltpu.VMEM((CH, D), jnp.bfloat16),
        pltpu.VMEM((CH, D), jnp.bfloat16),
        pltpu.VMEM((CH, D), jnp.bfloat16),
        pltpu.VMEM_SHARED((N, D), jnp.bfloat16),
        pltpu.SemaphoreType.DMA,
        pltpu.SemaphoreType.DMA,
    ],
)


RB = 2000


def _norm(a_ref, b_ref):
    return lax.rsqrt(a_ref[:, :1] + b_ref[:, :1] + 1.0)


def _norm(a_ref, b_ref):
    return lax.rsqrt(a_ref[:, :1] + b_ref[:, :1] + 1.0)


def _tc1_body(x_ref, ds0_ref, ds1_ref, w_ref, o_ref):
    ns = _norm(ds0_ref, ds1_ref)
    o_ref[...] = jnp.dot(x_ref[...] * ns, w_ref[...],
                         preferred_element_type=jnp.float32).astype(jnp.bfloat16)


def _tc2_body(p_ref, g_ref, dd0_ref, dd1_ref, ds0_ref, ds1_ref, b_ref, w_ref,
              o_ref):
    nd = _norm(dd0_ref, dd1_ref)
    ns = _norm(ds0_ref, ds1_ref)
    h = ((p_ref[0] + p_ref[1]).astype(jnp.float32)
         + g_ref[...].astype(jnp.float32)) * nd + b_ref[...]
    h = jnp.maximum(h, 0.0)
    o_ref[...] = jnp.dot(h * ns, w_ref[...],
                         preferred_element_type=jnp.float32).astype(jnp.bfloat16)


def _tc3_body(p_ref, g_ref, dd0_ref, dd1_ref, b_ref, o_ref):
    nd = _norm(dd0_ref, dd1_ref)
    o_ref[...] = ((p_ref[0] + p_ref[1]).astype(jnp.float32)
                  + g_ref[...].astype(jnp.float32)) * nd + b_ref[...]


def _deg_spec(off):
    return pl.BlockSpec((RB, DEG_W), lambda i, off=off: (i + off, 0))


_NBLOCKS = N // RB
_S0 = 0
_D0 = _NBLOCKS
_S1 = 2 * _NBLOCKS
_D1 = 3 * _NBLOCKS

_mat_spec = pl.BlockSpec((RB, D), lambda i: (i, 0))
_w_spec = pl.BlockSpec((D, D), lambda i: (0, 0))
_b_spec = pl.BlockSpec((1, D), lambda i: (0, 0))
_p_spec = pl.BlockSpec((2, RB, D), lambda i: (0, i, 0))

_tc1 = pl.pallas_call(
    _tc1_body,
    out_shape=jax.ShapeDtypeStruct((N, D), jnp.bfloat16),
    grid=(_NBLOCKS,),
    in_specs=[_mat_spec, _deg_spec(_S0), _deg_spec(_S1), _w_spec],
    out_specs=_mat_spec,
)

_tc2 = pl.pallas_call(
    _tc2_body,
    out_shape=jax.ShapeDtypeStruct((N, D), jnp.bfloat16),
    grid=(_NBLOCKS,),
    in_specs=[_p_spec, _mat_spec, _deg_spec(_D0), _deg_spec(_D1),
              _deg_spec(_S0), _deg_spec(_S1), _b_spec, _w_spec],
    out_specs=_mat_spec,
)

_tc3 = pl.pallas_call(
    _tc3_body,
    out_shape=jax.ShapeDtypeStruct((N, D), jnp.float32),
    grid=(_NBLOCKS,),
    in_specs=[_p_spec, _mat_spec, _deg_spec(_D0), _deg_spec(_D1), _b_spec],
    out_specs=_mat_spec,
)


def kernel(x, edge_index, W1, b1, W2, b2):
    er = edge_index.reshape(2, AGG_ROWS, CH)
    upd = jnp.zeros((CH, DEG_W), jnp.float32).at[:, 0].set(1.0)
    zdeg = jnp.zeros((ROWS_PER_TILE, DEG_W), jnp.float32)
    zagg = jnp.zeros((ROWS_PER_TILE, D), jnp.bfloat16)

    degp = _deg_call(er, upd, zdeg)
    g1 = _tc1(x, degp, degp, W1)
    p1 = _agg_call(g1, er, zagg)
    g2 = _tc2(p1.reshape(2, N, D), g1, degp, degp, degp, degp,
              b1.reshape(1, D), W2)
    p2 = _agg_call(g2, er, zagg)
    out = _tc3(p2.reshape(2, N, D), g2, degp, degp, b2.reshape(1, D))
    return out
